# baseline (speedup 1.0000x reference)
import functools

import jax
import jax.numpy as jnp
from jax import lax
from jax.experimental import pallas as pl
from jax.experimental.pallas import tpu as pltpu
from jax.experimental.pallas import tpu_sc as plsc

N = 10000
E = 320000
D_IN = 128
H = 256
HH = 128
K = 20
F_ = 16
L = 4
T = 12
B = 64

NT = 16
CH = 80
RPT = 624
TAIL0 = NT * RPT
TAILN = N - TAIL0

CHA = 128
NCHUNK = 160
EP = NT * NCHUNK * CHA
NBUF = 2
IB = 16
NBLK = NCHUNK // IB
NPAD = N + 8
EPC = E // 2
EPTD = EPC // NT

_f32 = jnp.float32


def _make_agg_kernel(W, tag):
    mesh = plsc.VectorSubcoreMesh(core_axis_name="c", subcore_axis_name="s")

    @functools.partial(
        pl.kernel,
        mesh=mesh,
        out_type=[
            jax.ShapeDtypeStruct((N, W), _f32),
            jax.ShapeDtypeStruct((N, W), _f32),
        ],
        scratch_types=[
            pltpu.VMEM((IB, CHA), jnp.int32),
            pltpu.VMEM((IB, CHA), jnp.int32),
            *[pltpu.VMEM((CHA, W), _f32) for _ in range(NBUF)],
            pltpu.VMEM_SHARED((NPAD, W), _f32),
            *[pltpu.SemaphoreType.DMA for _ in range(2 * NBUF)],
            pltpu.VMEM((8, 8 * (tag + 1)), jnp.int32),
        ],
    )
    def agg_kernel(hA_ref, hB_ref, src_ref, dst_ref, zer_ref,
                   outA, outB, srcv, dstv, *bufs_acc_sems):
        rows = bufs_acc_sems[:NBUF]
        acc = bufs_acc_sems[NBUF]
        sg = bufs_acc_sems[NBUF + 1:NBUF + 1 + NBUF]
        ss = bufs_acc_sems[NBUF + 1 + NBUF:NBUF + 1 + 2 * NBUF]
        c = lax.axis_index("c")
        s = lax.axis_index("s")
        r0 = s * RPT
        pltpu.sync_copy(zer_ref, acc.at[pl.ds(r0, RPT)])

        @pl.when(s == NT - 1)
        def _():
            pltpu.sync_copy(zer_ref.at[pl.ds(0, NPAD - TAIL0)],
                            acc.at[pl.ds(TAIL0, NPAD - TAIL0)])

        plsc.subcore_barrier()

        def run(table_ref):
            def gather(k, b):
                pltpu.async_copy(table_ref.at[srcv.at[k]], rows[b], sg[b])

            def gather_wait(b):
                pltpu.make_async_copy(table_ref.at[srcv.at[0]], rows[b],
                                      sg[b]).wait()

            def scat(k, b):
                pltpu.async_copy(rows[b], acc.at[dstv.at[k]], ss[b], add=True)

            def scat_wait(b):
                pltpu.make_async_copy(rows[b], acc.at[dstv.at[0]], ss[b]).wait()

            def body_blk(blk, carry):
                base_g = s * NCHUNK + blk * IB
                pltpu.sync_copy(src_ref.at[pl.ds(base_g, IB)], srcv)
                pltpu.sync_copy(dst_ref.at[pl.ds(base_g, IB)], dstv)
                for b in range(NBUF):
                    gather(b, b)

                def body(k4, cc):
                    base = k4 * NBUF
                    for b in range(NBUF):
                        gather_wait(b)
                        scat(base + b, b)

                    @pl.when(k4 < IB // NBUF - 1)
                    def _():
                        for b in range(NBUF):
                            scat_wait(b)
                            gather(base + NBUF + b, b)

                    return cc

                lax.fori_loop(0, IB // NBUF, body, 0)
                for b in range(NBUF):
                    scat_wait(b)
                return carry

            lax.fori_loop(0, NBLK, body_blk, 0)

        @pl.when(c == 0)
        def _():
            run(hA_ref)

        @pl.when(c == 1)
        def _():
            run(hB_ref)

        plsc.subcore_barrier()

        @pl.when(c == 0)
        def _():
            pltpu.sync_copy(acc.at[pl.ds(r0, RPT)], outA.at[pl.ds(r0, RPT)])

            @pl.when(s == NT - 1)
            def _():
                pltpu.sync_copy(acc.at[pl.ds(TAIL0, TAILN)],
                                outA.at[pl.ds(TAIL0, TAILN)])

        @pl.when(c == 1)
        def _():
            pltpu.sync_copy(acc.at[pl.ds(r0, RPT)], outB.at[pl.ds(r0, RPT)])

            @pl.when(s == NT - 1)
            def _():
                pltpu.sync_copy(acc.at[pl.ds(TAIL0, TAILN)],
                                outB.at[pl.ds(TAIL0, TAILN)])

    return agg_kernel


_agg_calls = [_make_agg_kernel(HH, t) for t in range(5)]


def _input_body(x_ref, w_ref, b_ref, dep_ref, outA_ref, outB_ref):
    del dep_ref
    h = jnp.dot(x_ref[...], w_ref[...], preferred_element_type=_f32)
    h = jnp.maximum(h + b_ref[...], 0.0)
    outA_ref[...] = h[:, :HH]
    outB_ref[...] = h[:, HH:]


_input_call = pl.pallas_call(
    _input_body,
    out_shape=(
        jax.ShapeDtypeStruct((N, HH), _f32),
        jax.ShapeDtypeStruct((N, HH), _f32),
    ),
)


def _layer_body(first, hA_ref, hB_ref, aggA_ref, aggB_ref, deg_ref,
                U_ref, Ws_ref, WnA_ref, WnB_ref, Wspec_ref, b_ref, theta_ref,
                aw_ref, ab_ref, bng_ref, bnb_ref,
                outA_ref, outB_ref):
    h = jnp.concatenate([hA_ref[...], hB_ref[...]], axis=1)
    invd = 1.0 / jnp.clip(deg_ref[:, :1], 1.0, None)
    spatial = jnp.dot(h, Ws_ref[...], preferred_element_type=_f32)
    spatial = spatial + jnp.dot(aggA_ref[...] * invd, WnA_ref[...],
                                preferred_element_type=_f32)
    spatial = spatial + jnp.dot(aggB_ref[...] * invd, WnB_ref[...],
                                preferred_element_type=_f32)
    U = U_ref[...]
    xs = lax.dot_general(U, h, (((0,), (0,)), ((), ())),
                         preferred_element_type=_f32)
    hmean = jnp.mean(h, axis=0, keepdims=True)
    g = jnp.dot(hmean, aw_ref[...], preferred_element_type=_f32) + ab_ref[...]
    g = g - jnp.max(g, axis=1, keepdims=True)
    eg = jnp.exp(g)
    gate = eg / jnp.sum(eg, axis=1, keepdims=True)
    filt = jnp.dot(gate, theta_ref[...], preferred_element_type=_f32)
    spec = jnp.dot(jnp.dot(U * filt, xs, preferred_element_type=_f32),
                   Wspec_ref[...], preferred_element_type=_f32)
    xn = spatial + spec + b_ref[...]
    mu = jnp.mean(xn, axis=0, keepdims=True)
    var = jnp.mean((xn - mu) * (xn - mu), axis=0, keepdims=True)
    xn = (xn - mu) * lax.rsqrt(var + 1e-5) * bng_ref[...] + bnb_ref[...]
    xn = jnp.maximum(xn, 0.0)
    hn = xn if first else h + xn
    outA_ref[...] = hn[:, :HH]
    outB_ref[...] = hn[:, HH:]


def _make_layer_call(first):
    return pl.pallas_call(
        functools.partial(_layer_body, first),
        out_shape=(
            jax.ShapeDtypeStruct((N, HH), _f32),
            jax.ShapeDtypeStruct((N, HH), _f32),
        ),
    )


_layer_first = _make_layer_call(True)
_layer_rest = _make_layer_call(False)


def _final_body(hA_ref, hB_ref, bid_ref, aw1_ref, ab1_ref, aw2_ref, ab2_ref,
                W1_ref, b1_ref, W2_ref, b2_ref, out_ref):
    h = jnp.concatenate([hA_ref[...], hB_ref[...]], axis=1)
    oh = (lax.broadcasted_iota(jnp.int32, (B, N), 0) == bid_ref[...]).astype(_f32)
    cnt = jnp.clip(jnp.sum(oh, axis=1, keepdims=True), 1.0, None)
    gsum = jnp.dot(oh, h, preferred_element_type=_f32)
    a = jnp.dot(jnp.tanh(jnp.dot(h, aw1_ref[...], preferred_element_type=_f32)
                         + ab1_ref[...]),
                aw2_ref[...], preferred_element_type=_f32) + ab2_ref[...]
    a = a - jnp.max(a, axis=0, keepdims=True)
    ea = jnp.exp(a)
    w = ea / jnp.sum(ea, axis=0, keepdims=True)
    gsum2 = jnp.dot(oh, h * w, preferred_element_type=_f32)
    gemb = (gsum + gsum2) / cnt
    hh = jnp.maximum(jnp.dot(gemb, W1_ref[...], preferred_element_type=_f32)
                     + b1_ref[...], 0.0)
    out_ref[...] = jnp.dot(hh, W2_ref[...], preferred_element_type=_f32) + b2_ref[...]


_final_call = pl.pallas_call(
    _final_body,
    out_shape=jax.ShapeDtypeStruct((B, T), _f32),
)


def kernel(x, edge_index, batch, laplacian_eigenvectors, params):
    src = edge_index[0].astype(jnp.int32)
    dst = edge_index[1].astype(jnp.int32)
    bid = batch.astype(jnp.int32).reshape(1, N)
    U = laplacian_eigenvectors

    zer = jnp.zeros((RPT, HH), _f32)
    pad = EP - E
    src2 = jnp.concatenate([src, jnp.zeros((pad,), jnp.int32)]).reshape(NT * NCHUNK, CHA)
    dst2 = jnp.concatenate([dst, jnp.full((pad,), N, jnp.int32)]).reshape(NT * NCHUNK, CHA)

    onesT = jnp.ones((N, HH), _f32)
    deg, _unused = _agg_calls[4](onesT, onesT, src2, dst2, zer)
    hA, hB = _input_call(x, params['input_proj']['w'],
                         params['input_proj']['b'].reshape(1, H),
                         deg[:1, :])

    for i, lp in enumerate(params['layers']):
        aggA, aggB = _agg_calls[i](hA, hB, src2, dst2, zer)
        call = _layer_first if i == 0 else _layer_rest
        hA, hB = call(
            hA, hB, aggA, aggB, deg, U,
            lp['W_s'], lp['W_n'][:HH], lp['W_n'][HH:], lp['W_spec'],
            lp['b'].reshape(1, H), lp['theta'], lp['adapt_w'],
            lp['adapt_b'].reshape(1, F_),
            lp['bn_g'].reshape(1, H), lp['bn_b'].reshape(1, H),
        )

    heads = params['heads']
    W1all = jnp.concatenate([hd['w1'] for hd in heads], axis=1)
    b1all = jnp.concatenate([hd['b1'] for hd in heads]).reshape(1, -1)
    W2blk = jnp.zeros((T * (H // 2), T), _f32)
    for t, hd in enumerate(heads):
        W2blk = W2blk.at[t * (H // 2):(t + 1) * (H // 2), t].set(hd['w2'][:, 0])
    b2all = jnp.concatenate([hd['b2'] for hd in heads]).reshape(1, T)

    attn = params['attn']
    out = _final_call(hA, hB, bid, attn['w1'], attn['b1'].reshape(1, H // 2),
                      attn['w2'], attn['b2'].reshape(1, 1),
                      W1all, b1all, W2blk, b2all)
    return out


# speedup vs baseline: 1.0134x; 1.0134x over previous
import functools

import jax
import jax.numpy as jnp
from jax import lax
from jax.experimental import pallas as pl
from jax.experimental.pallas import tpu as pltpu
from jax.experimental.pallas import tpu_sc as plsc

N = 10000
E = 320000
D_IN = 128
H = 256
HH = 128
K = 20
F_ = 16
L = 4
T = 12
B = 64

NT = 16
CH = 80
RPT = 624
TAIL0 = NT * RPT
TAILN = N - TAIL0

CHA = 128
NCHUNK = 160
EP = NT * NCHUNK * CHA
NBUF = 2
IB = 16
NBLK = NCHUNK // IB
NPAD = N + 8
EPC = E // 2
EPTD = EPC // NT

_f32 = jnp.float32


def _make_agg_kernel(W, tag):
    mesh = plsc.VectorSubcoreMesh(core_axis_name="c", subcore_axis_name="s")

    @functools.partial(
        pl.kernel,
        mesh=mesh,
        out_type=[
            jax.ShapeDtypeStruct((N, W), _f32),
            jax.ShapeDtypeStruct((N, W), _f32),
        ],
        scratch_types=[
            *[pltpu.VMEM((IB, CHA), jnp.int32) for _ in range(4)],
            *[pltpu.VMEM((CHA, W), _f32) for _ in range(NBUF)],
            pltpu.VMEM_SHARED((NPAD, W), _f32),
            *[pltpu.SemaphoreType.DMA for _ in range(2 * NBUF + 2)],
            pltpu.VMEM((8, 8 * (tag + 1)), jnp.int32),
        ],
    )
    def agg_kernel(hA_ref, hB_ref, src_ref, dst_ref, zer_ref,
                   outA, outB, sv0, sv1, dv0, dv1, *bufs_acc_sems):
        srcv2 = (sv0, sv1)
        dstv2 = (dv0, dv1)
        rows = bufs_acc_sems[:NBUF]
        acc = bufs_acc_sems[NBUF]
        sg = bufs_acc_sems[NBUF + 1:NBUF + 1 + NBUF]
        ss = bufs_acc_sems[NBUF + 1 + NBUF:NBUF + 1 + 2 * NBUF]
        si = bufs_acc_sems[NBUF + 1 + 2 * NBUF:NBUF + 3 + 2 * NBUF]
        c = lax.axis_index("c")
        s = lax.axis_index("s")
        r0 = s * RPT
        pltpu.sync_copy(zer_ref, acc.at[pl.ds(r0, RPT)])

        @pl.when(s == NT - 1)
        def _():
            pltpu.sync_copy(zer_ref.at[pl.ds(0, NPAD - TAIL0)],
                            acc.at[pl.ds(TAIL0, NPAD - TAIL0)])

        plsc.subcore_barrier()

        def run(table_ref):
            def gather(srcv, k, b):
                pltpu.async_copy(table_ref.at[srcv.at[k]], rows[b], sg[b])

            def gather_wait(srcv, b):
                pltpu.make_async_copy(table_ref.at[srcv.at[0]], rows[b],
                                      sg[b]).wait()

            def scat(dstv, k, b):
                pltpu.async_copy(rows[b], acc.at[dstv.at[k]], ss[b], add=True)

            def scat_wait(dstv, b):
                pltpu.make_async_copy(rows[b], acc.at[dstv.at[0]], ss[b]).wait()

            def pref(blk, p):
                base_g = s * NCHUNK + blk * IB
                pltpu.async_copy(src_ref.at[pl.ds(base_g, IB)], srcv2[p], si[0])
                pltpu.async_copy(dst_ref.at[pl.ds(base_g, IB)], dstv2[p], si[1])

            def pref_wait(blk, p):
                base_g = s * NCHUNK + blk * IB
                pltpu.make_async_copy(src_ref.at[pl.ds(base_g, IB)], srcv2[p],
                                      si[0]).wait()
                pltpu.make_async_copy(dst_ref.at[pl.ds(base_g, IB)], dstv2[p],
                                      si[1]).wait()

            pref(0, 0)
            pref_wait(0, 0)
            for blk in range(NBLK):
                cur = blk % 2
                srcv = srcv2[cur]
                dstv = dstv2[cur]
                if blk < NBLK - 1:
                    pref(blk + 1, 1 - cur)
                for b in range(NBUF):
                    gather(srcv, b, b)

                def body(k4, cc):
                    base = k4 * NBUF
                    for b in range(NBUF):
                        gather_wait(srcv, b)
                        scat(dstv, base + b, b)

                    @pl.when(k4 < IB // NBUF - 1)
                    def _():
                        for b in range(NBUF):
                            scat_wait(dstv, b)
                            gather(srcv, base + NBUF + b, b)

                    return cc

                lax.fori_loop(0, IB // NBUF, body, 0)
                for b in range(NBUF):
                    scat_wait(dstv, b)
                if blk < NBLK - 1:
                    pref_wait(blk + 1, 1 - cur)

        @pl.when(c == 0)
        def _():
            run(hA_ref)

        @pl.when(c == 1)
        def _():
            run(hB_ref)

        plsc.subcore_barrier()

        @pl.when(c == 0)
        def _():
            pltpu.sync_copy(acc.at[pl.ds(r0, RPT)], outA.at[pl.ds(r0, RPT)])

            @pl.when(s == NT - 1)
            def _():
                pltpu.sync_copy(acc.at[pl.ds(TAIL0, TAILN)],
                                outA.at[pl.ds(TAIL0, TAILN)])

        @pl.when(c == 1)
        def _():
            pltpu.sync_copy(acc.at[pl.ds(r0, RPT)], outB.at[pl.ds(r0, RPT)])

            @pl.when(s == NT - 1)
            def _():
                pltpu.sync_copy(acc.at[pl.ds(TAIL0, TAILN)],
                                outB.at[pl.ds(TAIL0, TAILN)])

    return agg_kernel


_agg_calls = [_make_agg_kernel(HH, t) for t in range(5)]


def _input_body(x_ref, w_ref, b_ref, dep_ref, outA_ref, outB_ref):
    del dep_ref
    h = jnp.dot(x_ref[...], w_ref[...], preferred_element_type=_f32)
    h = jnp.maximum(h + b_ref[...], 0.0)
    outA_ref[...] = h[:, :HH]
    outB_ref[...] = h[:, HH:]


_input_call = pl.pallas_call(
    _input_body,
    out_shape=(
        jax.ShapeDtypeStruct((N, HH), _f32),
        jax.ShapeDtypeStruct((N, HH), _f32),
    ),
)


def _layer_body(first, hA_ref, hB_ref, aggA_ref, aggB_ref, deg_ref,
                U_ref, Ws_ref, WnA_ref, WnB_ref, Wspec_ref, b_ref, theta_ref,
                aw_ref, ab_ref, bng_ref, bnb_ref,
                outA_ref, outB_ref):
    h = jnp.concatenate([hA_ref[...], hB_ref[...]], axis=1)
    invd = 1.0 / jnp.clip(deg_ref[:, :1], 1.0, None)
    spatial = jnp.dot(h, Ws_ref[...], preferred_element_type=_f32)
    spatial = spatial + jnp.dot(aggA_ref[...] * invd, WnA_ref[...],
                                preferred_element_type=_f32)
    spatial = spatial + jnp.dot(aggB_ref[...] * invd, WnB_ref[...],
                                preferred_element_type=_f32)
    U = U_ref[...]
    xs = lax.dot_general(U, h, (((0,), (0,)), ((), ())),
                         preferred_element_type=_f32)
    hmean = jnp.mean(h, axis=0, keepdims=True)
    g = jnp.dot(hmean, aw_ref[...], preferred_element_type=_f32) + ab_ref[...]
    g = g - jnp.max(g, axis=1, keepdims=True)
    eg = jnp.exp(g)
    gate = eg / jnp.sum(eg, axis=1, keepdims=True)
    filt = jnp.dot(gate, theta_ref[...], preferred_element_type=_f32)
    spec = jnp.dot(jnp.dot(U * filt, xs, preferred_element_type=_f32),
                   Wspec_ref[...], preferred_element_type=_f32)
    xn = spatial + spec + b_ref[...]
    mu = jnp.mean(xn, axis=0, keepdims=True)
    var = jnp.mean((xn - mu) * (xn - mu), axis=0, keepdims=True)
    xn = (xn - mu) * lax.rsqrt(var + 1e-5) * bng_ref[...] + bnb_ref[...]
    xn = jnp.maximum(xn, 0.0)
    hn = xn if first else h + xn
    outA_ref[...] = hn[:, :HH]
    outB_ref[...] = hn[:, HH:]


def _make_layer_call(first):
    return pl.pallas_call(
        functools.partial(_layer_body, first),
        out_shape=(
            jax.ShapeDtypeStruct((N, HH), _f32),
            jax.ShapeDtypeStruct((N, HH), _f32),
        ),
    )


_layer_first = _make_layer_call(True)
_layer_rest = _make_layer_call(False)


def _final_body(hA_ref, hB_ref, bid_ref, aw1_ref, ab1_ref, aw2_ref, ab2_ref,
                W1_ref, b1_ref, W2_ref, b2_ref, out_ref):
    h = jnp.concatenate([hA_ref[...], hB_ref[...]], axis=1)
    oh = (lax.broadcasted_iota(jnp.int32, (B, N), 0) == bid_ref[...]).astype(_f32)
    cnt = jnp.clip(jnp.sum(oh, axis=1, keepdims=True), 1.0, None)
    gsum = jnp.dot(oh, h, preferred_element_type=_f32)
    a = jnp.dot(jnp.tanh(jnp.dot(h, aw1_ref[...], preferred_element_type=_f32)
                         + ab1_ref[...]),
                aw2_ref[...], preferred_element_type=_f32) + ab2_ref[...]
    a = a - jnp.max(a, axis=0, keepdims=True)
    ea = jnp.exp(a)
    w = ea / jnp.sum(ea, axis=0, keepdims=True)
    gsum2 = jnp.dot(oh, h * w, preferred_element_type=_f32)
    gemb = (gsum + gsum2) / cnt
    hh = jnp.maximum(jnp.dot(gemb, W1_ref[...], preferred_element_type=_f32)
                     + b1_ref[...], 0.0)
    out_ref[...] = jnp.dot(hh, W2_ref[...], preferred_element_type=_f32) + b2_ref[...]


_final_call = pl.pallas_call(
    _final_body,
    out_shape=jax.ShapeDtypeStruct((B, T), _f32),
)


def kernel(x, edge_index, batch, laplacian_eigenvectors, params):
    src = edge_index[0].astype(jnp.int32)
    dst = edge_index[1].astype(jnp.int32)
    bid = batch.astype(jnp.int32).reshape(1, N)
    U = laplacian_eigenvectors

    zer = jnp.zeros((RPT, HH), _f32)
    pad = EP - E
    src2 = jnp.concatenate([src, jnp.zeros((pad,), jnp.int32)]).reshape(NT * NCHUNK, CHA)
    dst2 = jnp.concatenate([dst, jnp.full((pad,), N, jnp.int32)]).reshape(NT * NCHUNK, CHA)

    onesT = jnp.ones((N, HH), _f32)
    deg, _unused = _agg_calls[4](onesT, onesT, src2, dst2, zer)
    hA, hB = _input_call(x, params['input_proj']['w'],
                         params['input_proj']['b'].reshape(1, H),
                         deg[:1, :])

    for i, lp in enumerate(params['layers']):
        aggA, aggB = _agg_calls[i](hA, hB, src2, dst2, zer)
        call = _layer_first if i == 0 else _layer_rest
        hA, hB = call(
            hA, hB, aggA, aggB, deg, U,
            lp['W_s'], lp['W_n'][:HH], lp['W_n'][HH:], lp['W_spec'],
            lp['b'].reshape(1, H), lp['theta'], lp['adapt_w'],
            lp['adapt_b'].reshape(1, F_),
            lp['bn_g'].reshape(1, H), lp['bn_b'].reshape(1, H),
        )

    heads = params['heads']
    W1all = jnp.concatenate([hd['w1'] for hd in heads], axis=1)
    b1all = jnp.concatenate([hd['b1'] for hd in heads]).reshape(1, -1)
    W2blk = jnp.zeros((T * (H // 2), T), _f32)
    for t, hd in enumerate(heads):
        W2blk = W2blk.at[t * (H // 2):(t + 1) * (H // 2), t].set(hd['w2'][:, 0])
    b2all = jnp.concatenate([hd['b2'] for hd in heads]).reshape(1, T)

    attn = params['attn']
    out = _final_call(hA, hB, bid, attn['w1'], attn['b1'].reshape(1, H // 2),
                      attn['w2'], attn['b2'].reshape(1, 1),
                      W1all, b1all, W2blk, b2all)
    return out


# speedup vs baseline: 1.2156x; 1.1995x over previous
import functools

import jax
import jax.numpy as jnp
from jax import lax
from jax.experimental import pallas as pl
from jax.experimental.pallas import tpu as pltpu
from jax.experimental.pallas import tpu_sc as plsc

N = 10000
E = 320000
D_IN = 128
H = 256
HH = 128
K = 20
F_ = 16
L = 4
T = 12
B = 64

NT = 16
CH = 80
RPT = 624
TAIL0 = NT * RPT
TAILN = N - TAIL0

CHA = 128
NCHUNK = 160
EP = NT * NCHUNK * CHA
NBUF = 2
IB = 16
NBLK = NCHUNK // IB
NPAD = N + 8
NCHD = (NT * NCHUNK) // (2 * NT)

_f32 = jnp.float32


def _make_agg_kernel(W, tag):
    mesh = plsc.VectorSubcoreMesh(core_axis_name="c", subcore_axis_name="s")

    @functools.partial(
        pl.kernel,
        mesh=mesh,
        out_type=[
            jax.ShapeDtypeStruct((N, W), _f32),
            jax.ShapeDtypeStruct((N, W), _f32),
        ],
        scratch_types=[
            *[pltpu.VMEM((IB, CHA), jnp.int32) for _ in range(4)],
            *[pltpu.VMEM((CHA, W), _f32) for _ in range(NBUF)],
            pltpu.VMEM_SHARED((NPAD, W), _f32),
            *[pltpu.SemaphoreType.DMA for _ in range(2 * NBUF + 2)],
            pltpu.VMEM((8, 8 * (tag + 1)), jnp.int32),
        ],
    )
    def agg_kernel(hA_ref, hB_ref, src_ref, dst_ref, zer_ref,
                   outA, outB, sv0, sv1, dv0, dv1, *bufs_acc_sems):
        srcv2 = (sv0, sv1)
        dstv2 = (dv0, dv1)
        rows = bufs_acc_sems[:NBUF]
        acc = bufs_acc_sems[NBUF]
        sg = bufs_acc_sems[NBUF + 1:NBUF + 1 + NBUF]
        ss = bufs_acc_sems[NBUF + 1 + NBUF:NBUF + 1 + 2 * NBUF]
        si = bufs_acc_sems[NBUF + 1 + 2 * NBUF:NBUF + 3 + 2 * NBUF]
        c = lax.axis_index("c")
        s = lax.axis_index("s")
        r0 = s * RPT
        pltpu.sync_copy(zer_ref, acc.at[pl.ds(r0, RPT)])

        @pl.when(s == NT - 1)
        def _():
            pltpu.sync_copy(zer_ref.at[pl.ds(0, NPAD - TAIL0)],
                            acc.at[pl.ds(TAIL0, NPAD - TAIL0)])

        plsc.subcore_barrier()

        def run(table_ref):
            def gather(srcv, k, b):
                pltpu.async_copy(table_ref.at[srcv.at[k]], rows[b], sg[b])

            def gather_wait(srcv, b):
                pltpu.make_async_copy(table_ref.at[srcv.at[0]], rows[b],
                                      sg[b]).wait()

            def scat(dstv, k, b):
                pltpu.async_copy(rows[b], acc.at[dstv.at[k]], ss[b], add=True)

            def scat_wait(dstv, b):
                pltpu.make_async_copy(rows[b], acc.at[dstv.at[0]], ss[b]).wait()

            def pref(blk, p):
                base_g = s * NCHUNK + blk * IB
                pltpu.async_copy(src_ref.at[pl.ds(base_g, IB)], srcv2[p], si[0])
                pltpu.async_copy(dst_ref.at[pl.ds(base_g, IB)], dstv2[p], si[1])

            def pref_wait(blk, p):
                base_g = s * NCHUNK + blk * IB
                pltpu.make_async_copy(src_ref.at[pl.ds(base_g, IB)], srcv2[p],
                                      si[0]).wait()
                pltpu.make_async_copy(dst_ref.at[pl.ds(base_g, IB)], dstv2[p],
                                      si[1]).wait()

            pref(0, 0)
            pref_wait(0, 0)
            for blk in range(NBLK):
                cur = blk % 2
                srcv = srcv2[cur]
                dstv = dstv2[cur]
                if blk < NBLK - 1:
                    pref(blk + 1, 1 - cur)
                for b in range(NBUF):
                    gather(srcv, b, b)

                def body(k4, cc):
                    base = k4 * NBUF
                    for b in range(NBUF):
                        gather_wait(srcv, b)
                        scat(dstv, base + b, b)

                    @pl.when(k4 < IB // NBUF - 1)
                    def _():
                        for b in range(NBUF):
                            scat_wait(dstv, b)
                            gather(srcv, base + NBUF + b, b)

                    return cc

                lax.fori_loop(0, IB // NBUF, body, 0)
                for b in range(NBUF):
                    scat_wait(dstv, b)
                if blk < NBLK - 1:
                    pref_wait(blk + 1, 1 - cur)

        @pl.when(c == 0)
        def _():
            run(hA_ref)

        @pl.when(c == 1)
        def _():
            run(hB_ref)

        plsc.subcore_barrier()

        @pl.when(c == 0)
        def _():
            pltpu.sync_copy(acc.at[pl.ds(r0, RPT)], outA.at[pl.ds(r0, RPT)])

            @pl.when(s == NT - 1)
            def _():
                pltpu.sync_copy(acc.at[pl.ds(TAIL0, TAILN)],
                                outA.at[pl.ds(TAIL0, TAILN)])

        @pl.when(c == 1)
        def _():
            pltpu.sync_copy(acc.at[pl.ds(r0, RPT)], outB.at[pl.ds(r0, RPT)])

            @pl.when(s == NT - 1)
            def _():
                pltpu.sync_copy(acc.at[pl.ds(TAIL0, TAILN)],
                                outB.at[pl.ds(TAIL0, TAILN)])

    return agg_kernel


_agg_calls = [_make_agg_kernel(HH, t) for t in range(4)]


def _make_deg_kernel():
    mesh = plsc.VectorSubcoreMesh(core_axis_name="c", subcore_axis_name="s")

    @functools.partial(
        pl.kernel,
        mesh=mesh,
        out_type=[
            jax.ShapeDtypeStruct((N, HH), _f32),
            jax.ShapeDtypeStruct((N, HH), _f32),
        ],
        scratch_types=[
            pltpu.VMEM((NCHD, CHA), jnp.int32),
            pltpu.VMEM((CHA, HH), _f32),
            pltpu.VMEM((CHA, HH), _f32),
            pltpu.VMEM_SHARED((NPAD, HH), _f32),
            pltpu.SemaphoreType.DMA,
            pltpu.SemaphoreType.DMA,
            pltpu.VMEM((8, 8 * 5), jnp.int32),
        ],
    )
    def deg_kernel(dst_ref, ones_ref, zer_ref, outA, outB,
                   dstv, o0, o1, acc, s0, s1, _pad):
        ones_bufs = (o0, o1)
        ss = (s0, s1)
        c = lax.axis_index("c")
        s = lax.axis_index("s")
        r0 = s * RPT
        pltpu.sync_copy(zer_ref, acc.at[pl.ds(r0, RPT)])

        @pl.when(s == NT - 1)
        def _():
            pltpu.sync_copy(zer_ref.at[pl.ds(0, NPAD - TAIL0)],
                            acc.at[pl.ds(TAIL0, NPAD - TAIL0)])

        pltpu.sync_copy(ones_ref, o0)
        pltpu.sync_copy(ones_ref, o1)
        pltpu.sync_copy(
            dst_ref.at[pl.ds(c * (NT * NCHD) + s * NCHD, NCHD)], dstv)
        plsc.subcore_barrier()

        def scat(k, b):
            pltpu.async_copy(ones_bufs[b], acc.at[dstv.at[k]], ss[b], add=True)

        def scat_wait(b):
            pltpu.make_async_copy(ones_bufs[b], acc.at[dstv.at[0]],
                                  ss[b]).wait()

        def body(k2, cc):
            base = k2 * 2
            for b in range(2):
                @pl.when(k2 > 0)
                def _():
                    scat_wait(b)

                scat(base + b, b)
            return cc

        lax.fori_loop(0, NCHD // 2, body, 0)
        for b in range(2):
            scat_wait(b)

        plsc.subcore_barrier()

        @pl.when(c == 0)
        def _():
            pltpu.sync_copy(acc.at[pl.ds(r0, RPT)], outA.at[pl.ds(r0, RPT)])

            @pl.when(s == NT - 1)
            def _():
                pltpu.sync_copy(acc.at[pl.ds(TAIL0, TAILN)],
                                outA.at[pl.ds(TAIL0, TAILN)])

        @pl.when(c == 1)
        def _():
            pltpu.sync_copy(acc.at[pl.ds(r0, RPT)], outB.at[pl.ds(r0, RPT)])

            @pl.when(s == NT - 1)
            def _():
                pltpu.sync_copy(acc.at[pl.ds(TAIL0, TAILN)],
                                outB.at[pl.ds(TAIL0, TAILN)])

    return deg_kernel


_deg_call = _make_deg_kernel()


def _input_body(x_ref, w_ref, b_ref, dep_ref, outA_ref, outB_ref):
    del dep_ref
    h = jnp.dot(x_ref[...], w_ref[...], preferred_element_type=_f32)
    h = jnp.maximum(h + b_ref[...], 0.0)
    outA_ref[...] = h[:, :HH]
    outB_ref[...] = h[:, HH:]


_input_call = pl.pallas_call(
    _input_body,
    out_shape=(
        jax.ShapeDtypeStruct((N, HH), _f32),
        jax.ShapeDtypeStruct((N, HH), _f32),
    ),
)


def _layer_body(first, hA_ref, hB_ref, aggA_ref, aggB_ref, degA_ref, degB_ref,
                U_ref, Ws_ref, WnA_ref, WnB_ref, Wspec_ref, b_ref, theta_ref,
                aw_ref, ab_ref, bng_ref, bnb_ref,
                outA_ref, outB_ref):
    h = jnp.concatenate([hA_ref[...], hB_ref[...]], axis=1)
    invd = 1.0 / jnp.clip(degA_ref[:, :1] + degB_ref[:, :1], 1.0, None)
    spatial = jnp.dot(h, Ws_ref[...], preferred_element_type=_f32)
    spatial = spatial + jnp.dot(aggA_ref[...] * invd, WnA_ref[...],
                                preferred_element_type=_f32)
    spatial = spatial + jnp.dot(aggB_ref[...] * invd, WnB_ref[...],
                                preferred_element_type=_f32)
    U = U_ref[...]
    xs = lax.dot_general(U, h, (((0,), (0,)), ((), ())),
                         preferred_element_type=_f32)
    hmean = jnp.mean(h, axis=0, keepdims=True)
    g = jnp.dot(hmean, aw_ref[...], preferred_element_type=_f32) + ab_ref[...]
    g = g - jnp.max(g, axis=1, keepdims=True)
    eg = jnp.exp(g)
    gate = eg / jnp.sum(eg, axis=1, keepdims=True)
    filt = jnp.dot(gate, theta_ref[...], preferred_element_type=_f32)
    spec = jnp.dot(jnp.dot(U * filt, xs, preferred_element_type=_f32),
                   Wspec_ref[...], preferred_element_type=_f32)
    xn = spatial + spec + b_ref[...]
    mu = jnp.mean(xn, axis=0, keepdims=True)
    var = jnp.mean((xn - mu) * (xn - mu), axis=0, keepdims=True)
    xn = (xn - mu) * lax.rsqrt(var + 1e-5) * bng_ref[...] + bnb_ref[...]
    xn = jnp.maximum(xn, 0.0)
    hn = xn if first else h + xn
    outA_ref[...] = hn[:, :HH]
    outB_ref[...] = hn[:, HH:]


def _make_layer_call(first):
    return pl.pallas_call(
        functools.partial(_layer_body, first),
        out_shape=(
            jax.ShapeDtypeStruct((N, HH), _f32),
            jax.ShapeDtypeStruct((N, HH), _f32),
        ),
    )


_layer_first = _make_layer_call(True)
_layer_rest = _make_layer_call(False)


def _final_body(hA_ref, hB_ref, bid_ref, aw1_ref, ab1_ref, aw2_ref, ab2_ref,
                W1_ref, b1_ref, W2_ref, b2_ref, out_ref):
    h = jnp.concatenate([hA_ref[...], hB_ref[...]], axis=1)
    oh = (lax.broadcasted_iota(jnp.int32, (B, N), 0) == bid_ref[...]).astype(_f32)
    cnt = jnp.clip(jnp.sum(oh, axis=1, keepdims=True), 1.0, None)
    gsum = jnp.dot(oh, h, preferred_element_type=_f32)
    a = jnp.dot(jnp.tanh(jnp.dot(h, aw1_ref[...], preferred_element_type=_f32)
                         + ab1_ref[...]),
                aw2_ref[...], preferred_element_type=_f32) + ab2_ref[...]
    a = a - jnp.max(a, axis=0, keepdims=True)
    ea = jnp.exp(a)
    w = ea / jnp.sum(ea, axis=0, keepdims=True)
    gsum2 = jnp.dot(oh, h * w, preferred_element_type=_f32)
    gemb = (gsum + gsum2) / cnt
    hh = jnp.maximum(jnp.dot(gemb, W1_ref[...], preferred_element_type=_f32)
                     + b1_ref[...], 0.0)
    out_ref[...] = jnp.dot(hh, W2_ref[...], preferred_element_type=_f32) + b2_ref[...]


_final_call = pl.pallas_call(
    _final_body,
    out_shape=jax.ShapeDtypeStruct((B, T), _f32),
)


def kernel(x, edge_index, batch, laplacian_eigenvectors, params):
    src = edge_index[0].astype(jnp.int32)
    dst = edge_index[1].astype(jnp.int32)
    bid = batch.astype(jnp.int32).reshape(1, N)
    U = laplacian_eigenvectors

    zer = jnp.zeros((RPT, HH), _f32)
    pad = EP - E
    src2 = jnp.concatenate([src, jnp.zeros((pad,), jnp.int32)]).reshape(NT * NCHUNK, CHA)
    dst2 = jnp.concatenate([dst, jnp.full((pad,), N, jnp.int32)]).reshape(NT * NCHUNK, CHA)

    onesB = jnp.ones((CHA, HH), _f32)
    degA, degB = _deg_call(dst2, onesB, zer)
    hA, hB = _input_call(x, params['input_proj']['w'],
                         params['input_proj']['b'].reshape(1, H),
                         degA[:1, :])

    for i, lp in enumerate(params['layers']):
        aggA, aggB = _agg_calls[i](hA, hB, src2, dst2, zer)
        call = _layer_first if i == 0 else _layer_rest
        hA, hB = call(
            hA, hB, aggA, aggB, degA, degB, U,
            lp['W_s'], lp['W_n'][:HH], lp['W_n'][HH:], lp['W_spec'],
            lp['b'].reshape(1, H), lp['theta'], lp['adapt_w'],
            lp['adapt_b'].reshape(1, F_),
            lp['bn_g'].reshape(1, H), lp['bn_b'].reshape(1, H),
        )

    heads = params['heads']
    W1all = jnp.concatenate([hd['w1'] for hd in heads], axis=1)
    b1all = jnp.concatenate([hd['b1'] for hd in heads]).reshape(1, -1)
    W2blk = jnp.zeros((T * (H // 2), T), _f32)
    for t, hd in enumerate(heads):
        W2blk = W2blk.at[t * (H // 2):(t + 1) * (H // 2), t].set(hd['w2'][:, 0])
    b2all = jnp.concatenate([hd['b2'] for hd in heads]).reshape(1, T)

    attn = params['attn']
    out = _final_call(hA, hB, bid, attn['w1'], attn['b1'].reshape(1, H // 2),
                      attn['w2'], attn['b2'].reshape(1, 1),
                      W1all, b1all, W2blk, b2all)
    return out


# speedup vs baseline: 1.3174x; 1.0837x over previous
import functools

import jax
import jax.numpy as jnp
from jax import lax
from jax.experimental import pallas as pl
from jax.experimental.pallas import tpu as pltpu
from jax.experimental.pallas import tpu_sc as plsc

N = 10000
E = 320000
D_IN = 128
H = 256
HH = 128
K = 20
F_ = 16
L = 4
T = 12
B = 64

NT = 16
CH = 80
RPT = 624
TAIL0 = NT * RPT
TAILN = N - TAIL0

CHA = 64
NCHUNK = 320
EP = NT * NCHUNK * CHA
NBUF = 4
IB = 16
NBLK = NCHUNK // IB
NPAD = N + 8
NCHD = (NT * NCHUNK) // (2 * NT)

_f32 = jnp.float32


def _make_agg_kernel(W, tag):
    mesh = plsc.VectorSubcoreMesh(core_axis_name="c", subcore_axis_name="s")

    @functools.partial(
        pl.kernel,
        mesh=mesh,
        out_type=[
            jax.ShapeDtypeStruct((N, W), _f32),
            jax.ShapeDtypeStruct((N, W), _f32),
        ],
        scratch_types=[
            *[pltpu.VMEM((IB, CHA), jnp.int32) for _ in range(4)],
            *[pltpu.VMEM((CHA, W), _f32) for _ in range(NBUF)],
            pltpu.VMEM_SHARED((NPAD, W), _f32),
            *[pltpu.SemaphoreType.DMA for _ in range(2 * NBUF + 2)],
            pltpu.VMEM((8, 8 * (tag + 1)), jnp.int32),
        ],
    )
    def agg_kernel(hA_ref, hB_ref, src_ref, dst_ref, zer_ref,
                   outA, outB, sv0, sv1, dv0, dv1, *bufs_acc_sems):
        srcv2 = (sv0, sv1)
        dstv2 = (dv0, dv1)
        rows = bufs_acc_sems[:NBUF]
        acc = bufs_acc_sems[NBUF]
        sg = bufs_acc_sems[NBUF + 1:NBUF + 1 + NBUF]
        ss = bufs_acc_sems[NBUF + 1 + NBUF:NBUF + 1 + 2 * NBUF]
        si = bufs_acc_sems[NBUF + 1 + 2 * NBUF:NBUF + 3 + 2 * NBUF]
        c = lax.axis_index("c")
        s = lax.axis_index("s")
        r0 = s * RPT
        pltpu.sync_copy(zer_ref, acc.at[pl.ds(r0, RPT)])

        @pl.when(s == NT - 1)
        def _():
            pltpu.sync_copy(zer_ref.at[pl.ds(0, NPAD - TAIL0)],
                            acc.at[pl.ds(TAIL0, NPAD - TAIL0)])

        plsc.subcore_barrier()

        def run(table_ref):
            def gather(srcv, k, b):
                pltpu.async_copy(table_ref.at[srcv.at[k]], rows[b], sg[b])

            def gather_wait(srcv, b):
                pltpu.make_async_copy(table_ref.at[srcv.at[0]], rows[b],
                                      sg[b]).wait()

            def scat(dstv, k, b):
                pltpu.async_copy(rows[b], acc.at[dstv.at[k]], ss[b], add=True)

            def scat_wait(dstv, b):
                pltpu.make_async_copy(rows[b], acc.at[dstv.at[0]], ss[b]).wait()

            def pref(blk, p):
                base_g = s * NCHUNK + blk * IB
                pltpu.async_copy(src_ref.at[pl.ds(base_g, IB)], srcv2[p], si[0])
                pltpu.async_copy(dst_ref.at[pl.ds(base_g, IB)], dstv2[p], si[1])

            def pref_wait(blk, p):
                base_g = s * NCHUNK + blk * IB
                pltpu.make_async_copy(src_ref.at[pl.ds(base_g, IB)], srcv2[p],
                                      si[0]).wait()
                pltpu.make_async_copy(dst_ref.at[pl.ds(base_g, IB)], dstv2[p],
                                      si[1]).wait()

            pref(0, 0)
            pref_wait(0, 0)
            for blk in range(NBLK):
                cur = blk % 2
                srcv = srcv2[cur]
                dstv = dstv2[cur]
                if blk < NBLK - 1:
                    pref(blk + 1, 1 - cur)
                for b in range(NBUF):
                    gather(srcv, b, b)

                def body(k4, cc):
                    base = k4 * NBUF
                    for b in range(NBUF):
                        gather_wait(srcv, b)
                        scat(dstv, base + b, b)

                    @pl.when(k4 < IB // NBUF - 1)
                    def _():
                        for b in range(NBUF):
                            scat_wait(dstv, b)
                            gather(srcv, base + NBUF + b, b)

                    return cc

                lax.fori_loop(0, IB // NBUF, body, 0)
                for b in range(NBUF):
                    scat_wait(dstv, b)
                if blk < NBLK - 1:
                    pref_wait(blk + 1, 1 - cur)

        @pl.when(c == 0)
        def _():
            run(hA_ref)

        @pl.when(c == 1)
        def _():
            run(hB_ref)

        plsc.subcore_barrier()

        @pl.when(c == 0)
        def _():
            pltpu.sync_copy(acc.at[pl.ds(r0, RPT)], outA.at[pl.ds(r0, RPT)])

            @pl.when(s == NT - 1)
            def _():
                pltpu.sync_copy(acc.at[pl.ds(TAIL0, TAILN)],
                                outA.at[pl.ds(TAIL0, TAILN)])

        @pl.when(c == 1)
        def _():
            pltpu.sync_copy(acc.at[pl.ds(r0, RPT)], outB.at[pl.ds(r0, RPT)])

            @pl.when(s == NT - 1)
            def _():
                pltpu.sync_copy(acc.at[pl.ds(TAIL0, TAILN)],
                                outB.at[pl.ds(TAIL0, TAILN)])

    return agg_kernel


_agg_calls = [_make_agg_kernel(HH, t) for t in range(4)]


def _make_deg_kernel():
    mesh = plsc.VectorSubcoreMesh(core_axis_name="c", subcore_axis_name="s")

    @functools.partial(
        pl.kernel,
        mesh=mesh,
        out_type=[
            jax.ShapeDtypeStruct((N, HH), _f32),
            jax.ShapeDtypeStruct((N, HH), _f32),
        ],
        scratch_types=[
            pltpu.VMEM((NCHD, CHA), jnp.int32),
            pltpu.VMEM((CHA, HH), _f32),
            pltpu.VMEM((CHA, HH), _f32),
            pltpu.VMEM_SHARED((NPAD, HH), _f32),
            pltpu.SemaphoreType.DMA,
            pltpu.SemaphoreType.DMA,
            pltpu.VMEM((8, 8 * 5), jnp.int32),
        ],
    )
    def deg_kernel(dst_ref, ones_ref, zer_ref, outA, outB,
                   dstv, o0, o1, acc, s0, s1, _pad):
        ones_bufs = (o0, o1)
        ss = (s0, s1)
        c = lax.axis_index("c")
        s = lax.axis_index("s")
        r0 = s * RPT
        pltpu.sync_copy(zer_ref, acc.at[pl.ds(r0, RPT)])

        @pl.when(s == NT - 1)
        def _():
            pltpu.sync_copy(zer_ref.at[pl.ds(0, NPAD - TAIL0)],
                            acc.at[pl.ds(TAIL0, NPAD - TAIL0)])

        pltpu.sync_copy(ones_ref, o0)
        pltpu.sync_copy(ones_ref, o1)
        pltpu.sync_copy(
            dst_ref.at[pl.ds(c * (NT * NCHD) + s * NCHD, NCHD)], dstv)
        plsc.subcore_barrier()

        def scat(k, b):
            pltpu.async_copy(ones_bufs[b], acc.at[dstv.at[k]], ss[b], add=True)

        def scat_wait(b):
            pltpu.make_async_copy(ones_bufs[b], acc.at[dstv.at[0]],
                                  ss[b]).wait()

        def body(k2, cc):
            base = k2 * 2
            for b in range(2):
                @pl.when(k2 > 0)
                def _():
                    scat_wait(b)

                scat(base + b, b)
            return cc

        lax.fori_loop(0, NCHD // 2, body, 0)
        for b in range(2):
            scat_wait(b)

        plsc.subcore_barrier()

        @pl.when(c == 0)
        def _():
            pltpu.sync_copy(acc.at[pl.ds(r0, RPT)], outA.at[pl.ds(r0, RPT)])

            @pl.when(s == NT - 1)
            def _():
                pltpu.sync_copy(acc.at[pl.ds(TAIL0, TAILN)],
                                outA.at[pl.ds(TAIL0, TAILN)])

        @pl.when(c == 1)
        def _():
            pltpu.sync_copy(acc.at[pl.ds(r0, RPT)], outB.at[pl.ds(r0, RPT)])

            @pl.when(s == NT - 1)
            def _():
                pltpu.sync_copy(acc.at[pl.ds(TAIL0, TAILN)],
                                outB.at[pl.ds(TAIL0, TAILN)])

    return deg_kernel


_deg_call = _make_deg_kernel()


def _input_body(x_ref, w_ref, b_ref, dep_ref, outA_ref, outB_ref):
    del dep_ref
    h = jnp.dot(x_ref[...], w_ref[...], preferred_element_type=_f32)
    h = jnp.maximum(h + b_ref[...], 0.0)
    outA_ref[...] = h[:, :HH]
    outB_ref[...] = h[:, HH:]


_input_call = pl.pallas_call(
    _input_body,
    out_shape=(
        jax.ShapeDtypeStruct((N, HH), _f32),
        jax.ShapeDtypeStruct((N, HH), _f32),
    ),
)


def _layer_body(first, hA_ref, hB_ref, aggA_ref, aggB_ref, degA_ref, degB_ref,
                U_ref, Ws_ref, WnA_ref, WnB_ref, Wspec_ref, b_ref, theta_ref,
                aw_ref, ab_ref, bng_ref, bnb_ref,
                outA_ref, outB_ref):
    h = jnp.concatenate([hA_ref[...], hB_ref[...]], axis=1)
    invd = 1.0 / jnp.clip(degA_ref[:, :1] + degB_ref[:, :1], 1.0, None)
    spatial = jnp.dot(h, Ws_ref[...], preferred_element_type=_f32)
    spatial = spatial + jnp.dot(aggA_ref[...] * invd, WnA_ref[...],
                                preferred_element_type=_f32)
    spatial = spatial + jnp.dot(aggB_ref[...] * invd, WnB_ref[...],
                                preferred_element_type=_f32)
    U = U_ref[...]
    xs = lax.dot_general(U, h, (((0,), (0,)), ((), ())),
                         preferred_element_type=_f32)
    hmean = jnp.mean(h, axis=0, keepdims=True)
    g = jnp.dot(hmean, aw_ref[...], preferred_element_type=_f32) + ab_ref[...]
    g = g - jnp.max(g, axis=1, keepdims=True)
    eg = jnp.exp(g)
    gate = eg / jnp.sum(eg, axis=1, keepdims=True)
    filt = jnp.dot(gate, theta_ref[...], preferred_element_type=_f32)
    spec = jnp.dot(jnp.dot(U * filt, xs, preferred_element_type=_f32),
                   Wspec_ref[...], preferred_element_type=_f32)
    xn = spatial + spec + b_ref[...]
    mu = jnp.mean(xn, axis=0, keepdims=True)
    var = jnp.mean((xn - mu) * (xn - mu), axis=0, keepdims=True)
    xn = (xn - mu) * lax.rsqrt(var + 1e-5) * bng_ref[...] + bnb_ref[...]
    xn = jnp.maximum(xn, 0.0)
    hn = xn if first else h + xn
    outA_ref[...] = hn[:, :HH]
    outB_ref[...] = hn[:, HH:]


def _make_layer_call(first):
    return pl.pallas_call(
        functools.partial(_layer_body, first),
        out_shape=(
            jax.ShapeDtypeStruct((N, HH), _f32),
            jax.ShapeDtypeStruct((N, HH), _f32),
        ),
    )


_layer_first = _make_layer_call(True)
_layer_rest = _make_layer_call(False)


def _final_body(hA_ref, hB_ref, bid_ref, aw1_ref, ab1_ref, aw2_ref, ab2_ref,
                W1_ref, b1_ref, W2_ref, b2_ref, out_ref):
    h = jnp.concatenate([hA_ref[...], hB_ref[...]], axis=1)
    oh = (lax.broadcasted_iota(jnp.int32, (B, N), 0) == bid_ref[...]).astype(_f32)
    cnt = jnp.clip(jnp.sum(oh, axis=1, keepdims=True), 1.0, None)
    gsum = jnp.dot(oh, h, preferred_element_type=_f32)
    a = jnp.dot(jnp.tanh(jnp.dot(h, aw1_ref[...], preferred_element_type=_f32)
                         + ab1_ref[...]),
                aw2_ref[...], preferred_element_type=_f32) + ab2_ref[...]
    a = a - jnp.max(a, axis=0, keepdims=True)
    ea = jnp.exp(a)
    w = ea / jnp.sum(ea, axis=0, keepdims=True)
    gsum2 = jnp.dot(oh, h * w, preferred_element_type=_f32)
    gemb = (gsum + gsum2) / cnt
    hh = jnp.maximum(jnp.dot(gemb, W1_ref[...], preferred_element_type=_f32)
                     + b1_ref[...], 0.0)
    out_ref[...] = jnp.dot(hh, W2_ref[...], preferred_element_type=_f32) + b2_ref[...]


_final_call = pl.pallas_call(
    _final_body,
    out_shape=jax.ShapeDtypeStruct((B, T), _f32),
)


def kernel(x, edge_index, batch, laplacian_eigenvectors, params):
    src = edge_index[0].astype(jnp.int32)
    dst = edge_index[1].astype(jnp.int32)
    bid = batch.astype(jnp.int32).reshape(1, N)
    U = laplacian_eigenvectors

    zer = jnp.zeros((RPT, HH), _f32)
    pad = EP - E
    src2 = jnp.concatenate([src, jnp.zeros((pad,), jnp.int32)]).reshape(NT * NCHUNK, CHA)
    dst2 = jnp.concatenate([dst, jnp.full((pad,), N, jnp.int32)]).reshape(NT * NCHUNK, CHA)

    onesB = jnp.ones((CHA, HH), _f32)
    degA, degB = _deg_call(dst2, onesB, zer)
    hA, hB = _input_call(x, params['input_proj']['w'],
                         params['input_proj']['b'].reshape(1, H),
                         degA[:1, :])

    for i, lp in enumerate(params['layers']):
        aggA, aggB = _agg_calls[i](hA, hB, src2, dst2, zer)
        call = _layer_first if i == 0 else _layer_rest
        hA, hB = call(
            hA, hB, aggA, aggB, degA, degB, U,
            lp['W_s'], lp['W_n'][:HH], lp['W_n'][HH:], lp['W_spec'],
            lp['b'].reshape(1, H), lp['theta'], lp['adapt_w'],
            lp['adapt_b'].reshape(1, F_),
            lp['bn_g'].reshape(1, H), lp['bn_b'].reshape(1, H),
        )

    heads = params['heads']
    W1all = jnp.concatenate([hd['w1'] for hd in heads], axis=1)
    b1all = jnp.concatenate([hd['b1'] for hd in heads]).reshape(1, -1)
    W2blk = jnp.zeros((T * (H // 2), T), _f32)
    for t, hd in enumerate(heads):
        W2blk = W2blk.at[t * (H // 2):(t + 1) * (H // 2), t].set(hd['w2'][:, 0])
    b2all = jnp.concatenate([hd['b2'] for hd in heads]).reshape(1, T)

    attn = params['attn']
    out = _final_call(hA, hB, bid, attn['w1'], attn['b1'].reshape(1, H // 2),
                      attn['w2'], attn['b2'].reshape(1, 1),
                      W1all, b1all, W2blk, b2all)
    return out


# speedup vs baseline: 1.3365x; 1.0145x over previous
import functools

import jax
import jax.numpy as jnp
from jax import lax
from jax.experimental import pallas as pl
from jax.experimental.pallas import tpu as pltpu
from jax.experimental.pallas import tpu_sc as plsc

N = 10000
E = 320000
D_IN = 128
H = 256
HH = 128
K = 20
F_ = 16
L = 4
T = 12
B = 64

NT = 16
CH = 80
RPT = 624
TAIL0 = NT * RPT
TAILN = N - TAIL0

CHA = 64
NCHUNK = 320
EP = NT * NCHUNK * CHA
NBUF = 4
IB = 32
NBLK = NCHUNK // IB
NPAD = N + 8
NCHD = (NT * NCHUNK) // (2 * NT)

_f32 = jnp.float32


def _make_agg_kernel(W, tag):
    mesh = plsc.VectorSubcoreMesh(core_axis_name="c", subcore_axis_name="s")

    @functools.partial(
        pl.kernel,
        mesh=mesh,
        out_type=[
            jax.ShapeDtypeStruct((N, W), _f32),
            jax.ShapeDtypeStruct((N, W), _f32),
        ],
        scratch_types=[
            *[pltpu.VMEM((IB, CHA), jnp.int32) for _ in range(4)],
            *[pltpu.VMEM((CHA, W), _f32) for _ in range(NBUF)],
            pltpu.VMEM_SHARED((NPAD, W), _f32),
            *[pltpu.SemaphoreType.DMA for _ in range(2 * NBUF + 2)],
            pltpu.VMEM((8, 8 * (tag + 1)), jnp.int32),
        ],
    )
    def agg_kernel(hA_ref, hB_ref, src_ref, dst_ref, zer_ref,
                   outA, outB, sv0, sv1, dv0, dv1, *bufs_acc_sems):
        srcv2 = (sv0, sv1)
        dstv2 = (dv0, dv1)
        rows = bufs_acc_sems[:NBUF]
        acc = bufs_acc_sems[NBUF]
        sg = bufs_acc_sems[NBUF + 1:NBUF + 1 + NBUF]
        ss = bufs_acc_sems[NBUF + 1 + NBUF:NBUF + 1 + 2 * NBUF]
        si = bufs_acc_sems[NBUF + 1 + 2 * NBUF:NBUF + 3 + 2 * NBUF]
        c = lax.axis_index("c")
        s = lax.axis_index("s")
        r0 = s * RPT
        pltpu.sync_copy(zer_ref, acc.at[pl.ds(r0, RPT)])

        @pl.when(s == NT - 1)
        def _():
            pltpu.sync_copy(zer_ref.at[pl.ds(0, NPAD - TAIL0)],
                            acc.at[pl.ds(TAIL0, NPAD - TAIL0)])

        plsc.subcore_barrier()

        def run(table_ref):
            def gather(srcv, k, b):
                pltpu.async_copy(table_ref.at[srcv.at[k]], rows[b], sg[b])

            def gather_wait(srcv, b):
                pltpu.make_async_copy(table_ref.at[srcv.at[0]], rows[b],
                                      sg[b]).wait()

            def scat(dstv, k, b):
                pltpu.async_copy(rows[b], acc.at[dstv.at[k]], ss[b], add=True)

            def scat_wait(dstv, b):
                pltpu.make_async_copy(rows[b], acc.at[dstv.at[0]], ss[b]).wait()

            def pref(blk, p):
                base_g = s * NCHUNK + blk * IB
                pltpu.async_copy(src_ref.at[pl.ds(base_g, IB)], srcv2[p], si[0])
                pltpu.async_copy(dst_ref.at[pl.ds(base_g, IB)], dstv2[p], si[1])

            def pref_wait(blk, p):
                base_g = s * NCHUNK + blk * IB
                pltpu.make_async_copy(src_ref.at[pl.ds(base_g, IB)], srcv2[p],
                                      si[0]).wait()
                pltpu.make_async_copy(dst_ref.at[pl.ds(base_g, IB)], dstv2[p],
                                      si[1]).wait()

            pref(0, 0)
            pref_wait(0, 0)
            for blk in range(NBLK):
                cur = blk % 2
                srcv = srcv2[cur]
                dstv = dstv2[cur]
                if blk < NBLK - 1:
                    pref(blk + 1, 1 - cur)
                for b in range(NBUF):
                    gather(srcv, b, b)

                def body(k4, cc):
                    base = k4 * NBUF
                    for b in range(NBUF):
                        gather_wait(srcv, b)
                        scat(dstv, base + b, b)

                    @pl.when(k4 < IB // NBUF - 1)
                    def _():
                        for b in range(NBUF):
                            scat_wait(dstv, b)
                            gather(srcv, base + NBUF + b, b)

                    return cc

                lax.fori_loop(0, IB // NBUF, body, 0)
                for b in range(NBUF):
                    scat_wait(dstv, b)
                if blk < NBLK - 1:
                    pref_wait(blk + 1, 1 - cur)

        @pl.when(c == 0)
        def _():
            run(hA_ref)

        @pl.when(c == 1)
        def _():
            run(hB_ref)

        plsc.subcore_barrier()

        @pl.when(c == 0)
        def _():
            pltpu.sync_copy(acc.at[pl.ds(r0, RPT)], outA.at[pl.ds(r0, RPT)])

            @pl.when(s == NT - 1)
            def _():
                pltpu.sync_copy(acc.at[pl.ds(TAIL0, TAILN)],
                                outA.at[pl.ds(TAIL0, TAILN)])

        @pl.when(c == 1)
        def _():
            pltpu.sync_copy(acc.at[pl.ds(r0, RPT)], outB.at[pl.ds(r0, RPT)])

            @pl.when(s == NT - 1)
            def _():
                pltpu.sync_copy(acc.at[pl.ds(TAIL0, TAILN)],
                                outB.at[pl.ds(TAIL0, TAILN)])

    return agg_kernel


_agg_calls = [_make_agg_kernel(HH, t) for t in range(4)]


def _make_deg_kernel():
    mesh = plsc.VectorSubcoreMesh(core_axis_name="c", subcore_axis_name="s")

    @functools.partial(
        pl.kernel,
        mesh=mesh,
        out_type=[
            jax.ShapeDtypeStruct((N, HH), _f32),
            jax.ShapeDtypeStruct((N, HH), _f32),
        ],
        scratch_types=[
            pltpu.VMEM((NCHD, CHA), jnp.int32),
            pltpu.VMEM((CHA, HH), _f32),
            pltpu.VMEM((CHA, HH), _f32),
            pltpu.VMEM_SHARED((NPAD, HH), _f32),
            pltpu.SemaphoreType.DMA,
            pltpu.SemaphoreType.DMA,
            pltpu.VMEM((8, 8 * 5), jnp.int32),
        ],
    )
    def deg_kernel(dst_ref, ones_ref, zer_ref, outA, outB,
                   dstv, o0, o1, acc, s0, s1, _pad):
        ones_bufs = (o0, o1)
        ss = (s0, s1)
        c = lax.axis_index("c")
        s = lax.axis_index("s")
        r0 = s * RPT
        pltpu.sync_copy(zer_ref, acc.at[pl.ds(r0, RPT)])

        @pl.when(s == NT - 1)
        def _():
            pltpu.sync_copy(zer_ref.at[pl.ds(0, NPAD - TAIL0)],
                            acc.at[pl.ds(TAIL0, NPAD - TAIL0)])

        pltpu.sync_copy(ones_ref, o0)
        pltpu.sync_copy(ones_ref, o1)
        pltpu.sync_copy(
            dst_ref.at[pl.ds(c * (NT * NCHD) + s * NCHD, NCHD)], dstv)
        plsc.subcore_barrier()

        def scat(k, b):
            pltpu.async_copy(ones_bufs[b], acc.at[dstv.at[k]], ss[b], add=True)

        def scat_wait(b):
            pltpu.make_async_copy(ones_bufs[b], acc.at[dstv.at[0]],
                                  ss[b]).wait()

        def body(k2, cc):
            base = k2 * 2
            for b in range(2):
                @pl.when(k2 > 0)
                def _():
                    scat_wait(b)

                scat(base + b, b)
            return cc

        lax.fori_loop(0, NCHD // 2, body, 0)
        for b in range(2):
            scat_wait(b)

        plsc.subcore_barrier()

        @pl.when(c == 0)
        def _():
            pltpu.sync_copy(acc.at[pl.ds(r0, RPT)], outA.at[pl.ds(r0, RPT)])

            @pl.when(s == NT - 1)
            def _():
                pltpu.sync_copy(acc.at[pl.ds(TAIL0, TAILN)],
                                outA.at[pl.ds(TAIL0, TAILN)])

        @pl.when(c == 1)
        def _():
            pltpu.sync_copy(acc.at[pl.ds(r0, RPT)], outB.at[pl.ds(r0, RPT)])

            @pl.when(s == NT - 1)
            def _():
                pltpu.sync_copy(acc.at[pl.ds(TAIL0, TAILN)],
                                outB.at[pl.ds(TAIL0, TAILN)])

    return deg_kernel


_deg_call = _make_deg_kernel()


def _input_body(x_ref, w_ref, b_ref, dep_ref, outA_ref, outB_ref):
    del dep_ref
    h = jnp.dot(x_ref[...], w_ref[...], preferred_element_type=_f32)
    h = jnp.maximum(h + b_ref[...], 0.0)
    outA_ref[...] = h[:, :HH]
    outB_ref[...] = h[:, HH:]


_input_call = pl.pallas_call(
    _input_body,
    out_shape=(
        jax.ShapeDtypeStruct((N, HH), _f32),
        jax.ShapeDtypeStruct((N, HH), _f32),
    ),
)


def _layer_body(first, hA_ref, hB_ref, aggA_ref, aggB_ref, degA_ref, degB_ref,
                U_ref, Ws_ref, WnA_ref, WnB_ref, Wspec_ref, b_ref, theta_ref,
                aw_ref, ab_ref, bng_ref, bnb_ref,
                outA_ref, outB_ref):
    h = jnp.concatenate([hA_ref[...], hB_ref[...]], axis=1)
    invd = 1.0 / jnp.clip(degA_ref[:, :1] + degB_ref[:, :1], 1.0, None)
    spatial = jnp.dot(h, Ws_ref[...], preferred_element_type=_f32)
    spatial = spatial + jnp.dot(aggA_ref[...] * invd, WnA_ref[...],
                                preferred_element_type=_f32)
    spatial = spatial + jnp.dot(aggB_ref[...] * invd, WnB_ref[...],
                                preferred_element_type=_f32)
    U = U_ref[...]
    xs = lax.dot_general(U, h, (((0,), (0,)), ((), ())),
                         preferred_element_type=_f32)
    hmean = jnp.mean(h, axis=0, keepdims=True)
    g = jnp.dot(hmean, aw_ref[...], preferred_element_type=_f32) + ab_ref[...]
    g = g - jnp.max(g, axis=1, keepdims=True)
    eg = jnp.exp(g)
    gate = eg / jnp.sum(eg, axis=1, keepdims=True)
    filt = jnp.dot(gate, theta_ref[...], preferred_element_type=_f32)
    spec = jnp.dot(jnp.dot(U * filt, xs, preferred_element_type=_f32),
                   Wspec_ref[...], preferred_element_type=_f32)
    xn = spatial + spec + b_ref[...]
    mu = jnp.mean(xn, axis=0, keepdims=True)
    var = jnp.mean((xn - mu) * (xn - mu), axis=0, keepdims=True)
    xn = (xn - mu) * lax.rsqrt(var + 1e-5) * bng_ref[...] + bnb_ref[...]
    xn = jnp.maximum(xn, 0.0)
    hn = xn if first else h + xn
    outA_ref[...] = hn[:, :HH]
    outB_ref[...] = hn[:, HH:]


def _make_layer_call(first):
    return pl.pallas_call(
        functools.partial(_layer_body, first),
        out_shape=(
            jax.ShapeDtypeStruct((N, HH), _f32),
            jax.ShapeDtypeStruct((N, HH), _f32),
        ),
    )


_layer_first = _make_layer_call(True)
_layer_rest = _make_layer_call(False)


def _final_body(hA_ref, hB_ref, bid_ref, aw1_ref, ab1_ref, aw2_ref, ab2_ref,
                W1_ref, b1_ref, W2_ref, b2_ref, out_ref):
    h = jnp.concatenate([hA_ref[...], hB_ref[...]], axis=1)
    oh = (lax.broadcasted_iota(jnp.int32, (B, N), 0) == bid_ref[...]).astype(_f32)
    cnt = jnp.clip(jnp.sum(oh, axis=1, keepdims=True), 1.0, None)
    gsum = jnp.dot(oh, h, preferred_element_type=_f32)
    a = jnp.dot(jnp.tanh(jnp.dot(h, aw1_ref[...], preferred_element_type=_f32)
                         + ab1_ref[...]),
                aw2_ref[...], preferred_element_type=_f32) + ab2_ref[...]
    a = a - jnp.max(a, axis=0, keepdims=True)
    ea = jnp.exp(a)
    w = ea / jnp.sum(ea, axis=0, keepdims=True)
    gsum2 = jnp.dot(oh, h * w, preferred_element_type=_f32)
    gemb = (gsum + gsum2) / cnt
    hh = jnp.maximum(jnp.dot(gemb, W1_ref[...], preferred_element_type=_f32)
                     + b1_ref[...], 0.0)
    out_ref[...] = jnp.dot(hh, W2_ref[...], preferred_element_type=_f32) + b2_ref[...]


_final_call = pl.pallas_call(
    _final_body,
    out_shape=jax.ShapeDtypeStruct((B, T), _f32),
)


def kernel(x, edge_index, batch, laplacian_eigenvectors, params):
    src = edge_index[0].astype(jnp.int32)
    dst = edge_index[1].astype(jnp.int32)
    bid = batch.astype(jnp.int32).reshape(1, N)
    U = laplacian_eigenvectors

    zer = jnp.zeros((RPT, HH), _f32)
    pad = EP - E
    src2 = jnp.concatenate([src, jnp.zeros((pad,), jnp.int32)]).reshape(NT * NCHUNK, CHA)
    dst2 = jnp.concatenate([dst, jnp.full((pad,), N, jnp.int32)]).reshape(NT * NCHUNK, CHA)

    onesB = jnp.ones((CHA, HH), _f32)
    degA, degB = _deg_call(dst2, onesB, zer)
    hA, hB = _input_call(x, params['input_proj']['w'],
                         params['input_proj']['b'].reshape(1, H),
                         degA[:1, :])

    for i, lp in enumerate(params['layers']):
        aggA, aggB = _agg_calls[i](hA, hB, src2, dst2, zer)
        call = _layer_first if i == 0 else _layer_rest
        hA, hB = call(
            hA, hB, aggA, aggB, degA, degB, U,
            lp['W_s'], lp['W_n'][:HH], lp['W_n'][HH:], lp['W_spec'],
            lp['b'].reshape(1, H), lp['theta'], lp['adapt_w'],
            lp['adapt_b'].reshape(1, F_),
            lp['bn_g'].reshape(1, H), lp['bn_b'].reshape(1, H),
        )

    heads = params['heads']
    W1all = jnp.concatenate([hd['w1'] for hd in heads], axis=1)
    b1all = jnp.concatenate([hd['b1'] for hd in heads]).reshape(1, -1)
    W2blk = jnp.zeros((T * (H // 2), T), _f32)
    for t, hd in enumerate(heads):
        W2blk = W2blk.at[t * (H // 2):(t + 1) * (H // 2), t].set(hd['w2'][:, 0])
    b2all = jnp.concatenate([hd['b2'] for hd in heads]).reshape(1, T)

    attn = params['attn']
    out = _final_call(hA, hB, bid, attn['w1'], attn['b1'].reshape(1, H // 2),
                      attn['w2'], attn['b2'].reshape(1, 1),
                      W1all, b1all, W2blk, b2all)
    return out
